# baseline (speedup 1.0000x reference)
import functools

import jax
import jax.numpy as jnp
from jax import lax
from jax.experimental import pallas as pl
from jax.experimental.pallas import tpu as pltpu
from jax.experimental.pallas import tpu_sc as plsc


def kernel(x, pos_table):
    B, S, D = x.shape
    dtype = pos_table.dtype

    info = plsc.get_sparse_core_info()
    NC, NS = info.num_cores, info.num_subcores
    rows_per_sc = S // NC

    T_ROWS = 96
    SP_ROWS = rows_per_sc - (NS - 1) * T_ROWS
    T_CHUNK = 48
    SP_CHUNK = 152
    assert (NS - 1) * T_ROWS + SP_ROWS == rows_per_sc
    assert SP_CHUNK * 4 == SP_ROWS

    mesh = plsc.VectorSubcoreMesh(core_axis_name="c", subcore_axis_name="s")

    @functools.partial(
        pl.kernel,
        mesh=mesh,
        out_type=jax.ShapeDtypeStruct((B, S, D), dtype),
        scratch_types=[
            pltpu.VMEM((2, T_CHUNK, D), dtype),
            pltpu.VMEM_SHARED((2, SP_CHUNK, D), dtype),
            pltpu.SemaphoreType.DMA,
            pltpu.SemaphoreType.DMA,
        ],
    )
    def broadcast_rows(table_hbm, out_hbm, tbuf, sbuf, lsem, ssem):
        cid = lax.axis_index("c")
        sid = lax.axis_index("s")
        sc_row0 = cid * rows_per_sc

        def run_ring(row0, nchunks, chunk, buf):
            loads = [None] * nchunks
            stores = [None] * nchunks

            def start_load(c):
                loads[c] = pltpu.async_copy(
                    table_hbm.at[pl.ds(row0 + c * chunk, chunk)],
                    buf.at[c % 2],
                    lsem,
                )

            start_load(0)
            if nchunks > 1:
                start_load(1)
            for c in range(nchunks):
                loads[c].wait()
                stores[c] = [
                    pltpu.async_copy(
                        buf.at[c % 2],
                        out_hbm.at[b, pl.ds(row0 + c * chunk, chunk)],
                        ssem,
                    )
                    for b in range(B)
                ]
                n = c + 2
                if n < nchunks:
                    for h in stores[n - 2]:
                        h.wait()
                    stores[n - 2] = None
                    start_load(n)
            for st in stores:
                if st is not None:
                    for h in st:
                        h.wait()

        @pl.when(sid == 0)
        def _spmem_path():
            run_ring(sc_row0, SP_ROWS // SP_CHUNK, SP_CHUNK, sbuf)

        @pl.when(sid != 0)
        def _tile_path():
            row0 = sc_row0 + SP_ROWS + (sid - 1) * T_ROWS
            run_ring(row0, T_ROWS // T_CHUNK, T_CHUNK, tbuf)

    return broadcast_rows(pos_table)


# speedup vs baseline: 1.0012x; 1.0012x over previous
import functools

import jax
import jax.numpy as jnp
from jax import lax
from jax.experimental import pallas as pl
from jax.experimental.pallas import tpu as pltpu
from jax.experimental.pallas import tpu_sc as plsc


def kernel(x, pos_table):
    B, S, D = x.shape
    dtype = pos_table.dtype

    info = plsc.get_sparse_core_info()
    NC, NS = info.num_cores, info.num_subcores
    NW = NC * NS
    rows_per_w = S // NW
    BIG = 56
    chunk_rows = [BIG, BIG, rows_per_w - 2 * BIG]
    chunk_off = [0, BIG, 2 * BIG]
    nchunks = len(chunk_rows)
    NBUF = 2

    mesh = plsc.VectorSubcoreMesh(core_axis_name="c", subcore_axis_name="s")

    @functools.partial(
        pl.kernel,
        mesh=mesh,
        out_type=jax.ShapeDtypeStruct((B, S, D), dtype),
        scratch_types=[
            pltpu.VMEM((NBUF, BIG, D), dtype),
            pltpu.SemaphoreType.DMA,
            pltpu.SemaphoreType.DMA,
        ],
    )
    def broadcast_rows(table_hbm, out_hbm, buf, lsem, ssem):
        wid = lax.axis_index("s") * NC + lax.axis_index("c")
        row0 = wid * rows_per_w

        loads = [None] * nchunks
        stores = [None] * nchunks

        def start_load(c):
            loads[c] = pltpu.async_copy(
                table_hbm.at[pl.ds(row0 + chunk_off[c], chunk_rows[c])],
                buf.at[c % NBUF, pl.ds(0, chunk_rows[c])],
                lsem,
            )

        for n in range(min(NBUF, nchunks)):
            start_load(n)
        next_load = NBUF
        drained = [False] * nchunks
        for c in range(nchunks):
            loads[c].wait()
            stores[c] = [
                pltpu.async_copy(
                    buf.at[c % NBUF, pl.ds(0, chunk_rows[c])],
                    out_hbm.at[b, pl.ds(row0 + chunk_off[c], chunk_rows[c])],
                    ssem,
                )
                for b in range(B)
            ]
            if next_load < nchunks and c == next_load - 1:
                for h in stores[next_load - NBUF]:
                    h.wait()
                drained[next_load - NBUF] = True
                start_load(next_load)
                next_load += 1
        for c in range(nchunks):
            if not drained[c]:
                for h in stores[c]:
                    h.wait()

    return broadcast_rows(pos_table)
